# Initial kernel scaffold; baseline (speedup 1.0000x reference)
import jax
import jax.numpy as jnp
from jax.experimental import pallas as pl


def kernel(input_ids, word_embeddings):
    raise NotImplementedError("write your pallas kernel here")


# speedup vs baseline: 5.3908x; 5.3908x over previous
import functools

import jax
import jax.numpy as jnp
from jax import lax
from jax.experimental import pallas as pl
from jax.experimental.pallas import tpu as pltpu
from jax.experimental.pallas import tpu_sc as plsc

NC = 2
NS = 16
L = 16
NW = NC * NS
D = 4


def _make_lookup(n):
    per = n // NW
    ch = 6400
    nch = per // ch
    iters = ch // L
    mesh = plsc.VectorSubcoreMesh(
        core_axis_name="c", subcore_axis_name="s",
        num_cores=NC, num_subcores=NS)

    @functools.partial(
        pl.kernel,
        out_type=jax.ShapeDtypeStruct((n * D,), jnp.float32),
        mesh=mesh,
        scratch_types=[
            pltpu.VMEM((ch,), jnp.int32),
            pltpu.VMEM((ch * D,), jnp.float32),
            pltpu.VMEM((2 * D * L,), jnp.float32),
        ],
        compiler_params=pltpu.CompilerParams(needs_layout_passes=False),
    )
    def lookup(idx_hbm, splat_hbm, out_hbm, idx_v, out_v, spl_v):
        wid = lax.axis_index("s") * NC + lax.axis_index("c")
        base = wid * per
        pltpu.sync_copy(splat_hbm, spl_v)
        c = [spl_v[pl.ds(j * L, L)] for j in range(2 * D)]
        iota = lax.iota(jnp.int32, L)
        pos = [iota * D + k for k in range(D)]

        def chunk_body(cc, _):
            off = base + cc * ch
            pltpu.sync_copy(idx_hbm.at[pl.ds(off, ch)], idx_v)

            def it(i, _):
                v = idx_v[pl.ds(i * L, L)]
                m = v == 0
                pb = jnp.broadcast_to(i * (L * D), (L,))
                for k in range(D):
                    w = jnp.where(m, c[k], c[D + k])
                    plsc.store_scatter(out_v, [pb + pos[k]], w)
                return 0

            lax.fori_loop(0, iters, it, 0, unroll=2)
            pltpu.sync_copy(out_v, out_hbm.at[pl.ds(off * D, ch * D)])
            return 0

        lax.fori_loop(0, nch, chunk_body, 0)

    return lookup


def kernel(input_ids, word_embeddings):
    b, s = input_ids.shape
    v, d = word_embeddings.shape
    n = b * s
    idx_flat = input_ids.reshape(n)
    splats = jnp.broadcast_to(
        word_embeddings.reshape(v * d, 1), (v * d, L)).reshape(v * d * L)
    out_flat = _make_lookup(n)(idx_flat, splats)
    return out_flat.reshape(b, s, d)


# speedup vs baseline: 5.4352x; 1.0082x over previous
import functools

import jax
import jax.numpy as jnp
from jax import lax
from jax.experimental import pallas as pl
from jax.experimental.pallas import tpu as pltpu
from jax.experimental.pallas import tpu_sc as plsc

NC = 2
NS = 16
L = 16
NW = NC * NS
D = 4


def _make_lookup(n):
    per = n // NW
    ch = 6400
    nch = per // ch
    iters = ch // L
    mesh = plsc.VectorSubcoreMesh(
        core_axis_name="c", subcore_axis_name="s",
        num_cores=NC, num_subcores=NS)

    @functools.partial(
        pl.kernel,
        out_type=jax.ShapeDtypeStruct((n * D,), jnp.float32),
        mesh=mesh,
        scratch_types=[
            pltpu.VMEM((ch,), jnp.int32),
            pltpu.VMEM((ch * D,), jnp.float32),
            pltpu.VMEM((2 * D * L,), jnp.float32),
        ],
        compiler_params=pltpu.CompilerParams(needs_layout_passes=False),
    )
    def lookup(idx_hbm, splat_hbm, out_hbm, idx_v, out_v, spl_v):
        wid = lax.axis_index("s") * NC + lax.axis_index("c")
        base = wid * per
        pltpu.sync_copy(splat_hbm, spl_v)
        c = [spl_v[pl.ds(j * L, L)] for j in range(2 * D)]
        iota = lax.iota(jnp.int32, L)
        pos = [iota * D + k for k in range(D)]

        def chunk_body(cc, _):
            off = base + cc * ch
            pltpu.sync_copy(idx_hbm.at[pl.ds(off, ch)], idx_v)

            @plsc.parallel_loop(0, iters, 1, unroll=8)
            def it(i):
                v = idx_v[pl.ds(i * L, L)]
                m = v == 0
                sl = out_v.at[pl.ds(i * (L * D), L * D)]
                for k in range(D):
                    w = jnp.where(m, c[k], c[D + k])
                    plsc.store_scatter(sl, [pos[k]], w)

            pltpu.sync_copy(out_v, out_hbm.at[pl.ds(off * D, ch * D)])
            return 0

        lax.fori_loop(0, nch, chunk_body, 0)

    return lookup


def kernel(input_ids, word_embeddings):
    b, s = input_ids.shape
    v, d = word_embeddings.shape
    n = b * s
    idx_flat = input_ids.reshape(n)
    splats = jnp.broadcast_to(
        word_embeddings.reshape(v * d, 1), (v * d, L)).reshape(v * d * L)
    out_flat = _make_lookup(n)(idx_flat, splats)
    return out_flat.reshape(b, s, d)


# speedup vs baseline: 269.7801x; 49.6361x over previous
import functools

import jax
import jax.numpy as jnp
from jax import lax
from jax.experimental import pallas as pl
from jax.experimental.pallas import tpu as pltpu
from jax.experimental.pallas import tpu_sc as plsc

NC = 2
NS = 16
L = 16
NW = NC * NS
D = 4

LB = 25
IN_U = 4096
OUT_U = 16384
OUT_BLK = 2048


def _make_lookup():
    mesh = plsc.VectorSubcoreMesh(
        core_axis_name="c", subcore_axis_name="s",
        num_cores=NC, num_subcores=NS)

    @functools.partial(
        pl.kernel,
        out_type=jax.ShapeDtypeStruct((NW * LB * OUT_U,), jnp.float32),
        mesh=mesh,
        scratch_types=[
            pltpu.VMEM((IN_U,), jnp.int32),
            pltpu.VMEM((IN_U,), jnp.int32),
            pltpu.VMEM((OUT_U,), jnp.float32),
            pltpu.VMEM((OUT_U,), jnp.float32),
            pltpu.VMEM((2 * D * L,), jnp.float32),
            pltpu.SemaphoreType.DMA,
            pltpu.SemaphoreType.DMA,
            pltpu.SemaphoreType.DMA,
            pltpu.SemaphoreType.DMA,
        ],
        compiler_params=pltpu.CompilerParams(needs_layout_passes=False),
    )
    def lookup(idx_hbm, spl_hbm, out_hbm,
               in_v0, in_v1, out_v0, out_v1, spl_v, si0, si1, so0, so1):
        wid = lax.axis_index("s") * NC + lax.axis_index("c")
        ibase = wid * IN_U
        obase = wid * OUT_BLK
        pltpu.sync_copy(spl_hbm, spl_v)
        c = [spl_v[pl.ds(j * L, L)] for j in range(2 * D)]
        inb = (in_v0, in_v1)
        outb = (out_v0, out_v1)
        sis = (si0, si1)
        sos = (so0, so1)

        def in_slice(u):
            return idx_hbm.at[pl.ds(u * (NW * IN_U) + ibase, IN_U)]

        def start_in(u, b):
            pltpu.async_copy(in_slice(u), inb[b], sis[b])

        def wait_in(u, b):
            pltpu.make_async_copy(in_slice(u), inb[b], sis[b]).wait()

        def out_pair(u, b, lr):
            src = outb[b].at[pl.ds(lr * OUT_BLK, OUT_BLK)]
            dst = out_hbm.at[pl.ds((u * 8 + lr) * (NW * OUT_BLK) + obase,
                                   OUT_BLK)]
            return src, dst

        def start_out(u, b):
            for lr in range(8):
                src, dst = out_pair(u, b, lr)
                pltpu.async_copy(src, dst, sos[b])

        def wait_out(u, b):
            for lr in range(8):
                src, dst = out_pair(u, b, lr)
                pltpu.make_async_copy(src, dst, sos[b]).wait()

        def compute(b):
            ib = inb[b]
            ob = outb[b]

            @plsc.parallel_loop(0, IN_U // L, 1, unroll=4)
            def q_loop(q):
                v = ib[pl.ds(q * L, L)]
                m = v == 0
                oo = ((q >> 3) & 7) * OUT_BLK + (q >> 6) * 512 + (q & 7) * L
                for k in range(D):
                    ob[pl.ds(oo + k * 128, L)] = jnp.where(m, c[k], c[D + k])

        start_in(0, 0)
        start_in(1, 1)
        wait_in(0, 0)
        compute(0)
        start_out(0, 0)
        start_in(2, 0)
        wait_in(1, 1)
        compute(1)
        start_out(1, 1)
        start_in(3, 1)

        def outer(uu, _):
            u0 = 2 * uu + 2
            for b in (0, 1):
                u = u0 + b
                wait_in(u, b)
                wait_out(u - 2, b)
                compute(b)
                start_out(u, b)

                @pl.when(u + 2 <= LB - 1)
                def _():
                    start_in(u + 2, b)

                del _
            return 0

        lax.fori_loop(0, (LB - 3) // 2, outer, 0)
        wait_in(LB - 1, 0)
        wait_out(LB - 3, 0)
        compute(0)
        start_out(LB - 1, 0)
        wait_out(LB - 2, 1)
        wait_out(LB - 1, 0)

    return lookup


def kernel(input_ids, word_embeddings):
    b, s = input_ids.shape
    v, d = word_embeddings.shape
    in_flat = (input_ids.T.reshape(s // 8, 8, b // 128, 128)
               .transpose(0, 2, 1, 3).reshape(b * s))
    splats = jnp.broadcast_to(
        word_embeddings.reshape(v * d, 1), (v * d, L)).reshape(v * d * L)
    out_flat = _make_lookup()(in_flat, splats)
    return (out_flat.reshape(s, b // 128, d, 128)
            .transpose(1, 3, 0, 2).reshape(b, s, d))


# speedup vs baseline: 273.4992x; 1.0138x over previous
import functools

import jax
import jax.numpy as jnp
from jax import lax
from jax.experimental import pallas as pl
from jax.experimental.pallas import tpu as pltpu
from jax.experimental.pallas import tpu_sc as plsc

NC = 2
NS = 16
L = 16
NW = NC * NS
D = 4

LB = 25
IN_U = 4096


def _make_lookup():
    mesh = plsc.VectorSubcoreMesh(
        core_axis_name="c", subcore_axis_name="s",
        num_cores=NC, num_subcores=NS)

    @functools.partial(
        pl.kernel,
        out_type=jax.ShapeDtypeStruct((8 * LB, 4 * NW * D, 128), jnp.float32),
        mesh=mesh,
        scratch_types=[
            pltpu.VMEM((IN_U,), jnp.int32),
            pltpu.VMEM((IN_U,), jnp.int32),
            pltpu.VMEM((8, 4 * D, 128), jnp.float32),
            pltpu.VMEM((8, 4 * D, 128), jnp.float32),
            pltpu.VMEM((2 * D * L,), jnp.float32),
            pltpu.SemaphoreType.DMA,
            pltpu.SemaphoreType.DMA,
            pltpu.SemaphoreType.DMA,
            pltpu.SemaphoreType.DMA,
        ],
        compiler_params=pltpu.CompilerParams(needs_layout_passes=False),
    )
    def lookup(idx_hbm, spl_hbm, out_hbm,
               in_v0, in_v1, out_v0, out_v1, spl_v, si0, si1, so0, so1):
        wid = lax.axis_index("s") * NC + lax.axis_index("c")
        ibase = wid * IN_U
        obase = wid * (4 * D)
        pltpu.sync_copy(spl_hbm, spl_v)
        c = [spl_v[pl.ds(j * L, L)] for j in range(2 * D)]
        inb = (in_v0, in_v1)
        outb = (out_v0, out_v1)
        sis = (si0, si1)
        sos = (so0, so1)

        def in_slice(u):
            return idx_hbm.at[pl.ds(u * (NW * IN_U) + ibase, IN_U)]

        def start_in(u, b):
            pltpu.async_copy(in_slice(u), inb[b], sis[b])

        def wait_in(u, b):
            pltpu.make_async_copy(in_slice(u), inb[b], sis[b]).wait()

        def out_slice(u):
            return out_hbm.at[pl.ds(u * 8, 8), pl.ds(obase, 4 * D), :]

        def start_out(u, b):
            pltpu.async_copy(outb[b], out_slice(u), sos[b])

        def wait_out(u, b):
            pltpu.make_async_copy(outb[b], out_slice(u), sos[b]).wait()

        def compute(b):
            ib = inb[b]
            ob = outb[b]

            @plsc.parallel_loop(0, IN_U // L, 1, unroll=4)
            def q_loop(q):
                v = ib[pl.ds(q * L, L)]
                m = v == 0
                bti = q >> 6
                lr = (q >> 3) & 7
                j = q & 7
                for k in range(D):
                    ob[lr, bti * D + k, pl.ds(j * L, L)] = (
                        jnp.where(m, c[k], c[D + k]))

        start_in(0, 0)
        start_in(1, 1)
        wait_in(0, 0)
        compute(0)
        start_out(0, 0)
        start_in(2, 0)
        wait_in(1, 1)
        compute(1)
        start_out(1, 1)
        start_in(3, 1)

        def outer(uu, _):
            u0 = 2 * uu + 2
            for b in (0, 1):
                u = u0 + b
                wait_in(u, b)
                wait_out(u - 2, b)
                compute(b)
                start_out(u, b)

                @pl.when(u + 2 <= LB - 1)
                def _():
                    start_in(u + 2, b)

                del _
            return 0

        lax.fori_loop(0, (LB - 3) // 2, outer, 0)
        wait_in(LB - 1, 0)
        wait_out(LB - 3, 0)
        compute(0)
        start_out(LB - 1, 0)
        wait_out(LB - 2, 1)
        wait_out(LB - 1, 0)

    return lookup


def kernel(input_ids, word_embeddings):
    b, s = input_ids.shape
    v, d = word_embeddings.shape
    in_flat = (input_ids.T.reshape(s // 8, 8, b // 128, 128)
               .transpose(0, 2, 1, 3).reshape(b * s))
    splats = jnp.broadcast_to(
        word_embeddings.reshape(v * d, 1), (v * d, L)).reshape(v * d * L)
    out3 = _make_lookup()(in_flat, splats)
    return (out3.reshape(s, b // 128, d, 128)
            .transpose(1, 3, 0, 2).reshape(b, s, d))


# speedup vs baseline: 280.7533x; 1.0265x over previous
import functools

import jax
import jax.numpy as jnp
from jax import lax
from jax.experimental import pallas as pl
from jax.experimental.pallas import tpu as pltpu
from jax.experimental.pallas import tpu_sc as plsc

NC = 2
NS = 16
L = 16
NW = NC * NS
D = 4

LB = 25
IN_U = 4096


def _make_lookup():
    mesh = plsc.VectorSubcoreMesh(
        core_axis_name="c", subcore_axis_name="s",
        num_cores=NC, num_subcores=NS)

    @functools.partial(
        pl.kernel,
        out_type=jax.ShapeDtypeStruct((8 * LB, 4 * NW * D, 128), jnp.float32),
        mesh=mesh,
        scratch_types=[
            pltpu.VMEM((IN_U,), jnp.int32),
            pltpu.VMEM((IN_U,), jnp.int32),
            pltpu.VMEM((IN_U,), jnp.int32),
            pltpu.VMEM((8, 4 * D, 128), jnp.float32),
            pltpu.VMEM((8, 4 * D, 128), jnp.float32),
            pltpu.VMEM((8, 4 * D, 128), jnp.float32),
            pltpu.VMEM((2 * D * L,), jnp.float32),
            pltpu.SemaphoreType.DMA,
            pltpu.SemaphoreType.DMA,
            pltpu.SemaphoreType.DMA,
            pltpu.SemaphoreType.DMA,
            pltpu.SemaphoreType.DMA,
            pltpu.SemaphoreType.DMA,
        ],
        compiler_params=pltpu.CompilerParams(needs_layout_passes=False),
    )
    def lookup(idx_hbm, spl_hbm, out_hbm,
               in_v0, in_v1, in_v2, out_v0, out_v1, out_v2, spl_v,
               si0, si1, si2, so0, so1, so2):
        wid = lax.axis_index("s") * NC + lax.axis_index("c")
        ibase = wid * IN_U
        obase = wid * (4 * D)
        pltpu.sync_copy(spl_hbm, spl_v)
        c = [spl_v[pl.ds(j * L, L)] for j in range(2 * D)]
        inb = (in_v0, in_v1, in_v2)
        outb = (out_v0, out_v1, out_v2)
        sis = (si0, si1, si2)
        sos = (so0, so1, so2)

        def in_slice(u):
            return idx_hbm.at[pl.ds(u * (NW * IN_U) + ibase, IN_U)]

        def start_in(u, b):
            pltpu.async_copy(in_slice(u), inb[b], sis[b])

        def wait_in(u, b):
            pltpu.make_async_copy(in_slice(u), inb[b], sis[b]).wait()

        def out_slice(u):
            return out_hbm.at[pl.ds(u * 8, 8), pl.ds(obase, 4 * D), :]

        def start_out(u, b):
            pltpu.async_copy(outb[b], out_slice(u), sos[b])

        def wait_out(u, b):
            pltpu.make_async_copy(outb[b], out_slice(u), sos[b]).wait()

        def compute(b):
            ib = inb[b]
            ob = outb[b]

            @plsc.parallel_loop(0, IN_U // L, 1, unroll=4)
            def q_loop(q):
                v = ib[pl.ds(q * L, L)]
                m = v == 0
                bti = q >> 6
                lr = (q >> 3) & 7
                j = q & 7
                for k in range(D):
                    ob[lr, bti * D + k, pl.ds(j * L, L)] = (
                        jnp.where(m, c[k], c[D + k]))

        for u in (0, 1, 2):
            start_in(u, u)
        for b in (0, 1, 2):
            wait_in(b, b)
            compute(b)
            start_out(b, b)
            start_in(b + 3, b)

        def outer(uu, _):
            u0 = 3 * uu + 3
            for b in (0, 1, 2):
                u = u0 + b
                wait_in(u, b)
                wait_out(u - 3, b)
                compute(b)
                start_out(u, b)

                @pl.when(u + 3 <= LB - 1)
                def _():
                    start_in(u + 3, b)

                del _
            return 0

        lax.fori_loop(0, (LB - 4) // 3, outer, 0)
        wait_in(LB - 1, 0)
        wait_out(LB - 4, 0)
        compute(0)
        start_out(LB - 1, 0)
        wait_out(LB - 3, 1)
        wait_out(LB - 2, 2)
        wait_out(LB - 1, 0)

    return lookup


def kernel(input_ids, word_embeddings):
    b, s = input_ids.shape
    v, d = word_embeddings.shape
    in_flat = (input_ids.T.reshape(s // 8, 8, b // 128, 128)
               .transpose(0, 2, 1, 3).reshape(b * s))
    splats = jnp.broadcast_to(
        word_embeddings.reshape(v * d, 1), (v * d, L)).reshape(v * d * L)
    out3 = _make_lookup()(in_flat, splats)
    return (out3.reshape(s, b // 128, d, 128)
            .transpose(1, 3, 0, 2).reshape(b, s, d))
